# baseline (speedup 1.0000x reference)
import functools

import jax
import jax.numpy as jnp
from jax import lax
from jax.experimental import pallas as pl
from jax.experimental.pallas import tpu as pltpu
from jax.experimental.pallas import tpu_sc as plsc

N = 4096 * 20
NC, NS, LANES = 2, 16, 16
NW = NC * NS
RPW = N // NW
CHUNK = 256
IDX_ROW = 128
G = CHUNK // IDX_ROW
NCHUNK = RPW // CHUNK
D = 64
VOC_BIG = 100001
VOC_4 = 1001


def _sc_gather_body(x_hbm, e0, e1, e4, o0, o1, o4,
                    xv, i0, i1, i4, rbufs, gsem, wsem):
    wid = lax.axis_index("s") * NC + lax.axis_index("c")
    lane = lax.iota(jnp.int32, LANES)

    def one_chunk(ci, p, guard):
        r0, r1, r4 = rbufs[p]
        @pl.when(guard)
        def _():
            prev = pl.ds((ci - 2) * CHUNK + wid * RPW, CHUNK)
            pltpu.make_async_copy(r0, o0.at[prev], wsem).wait()
            pltpu.make_async_copy(r1, o1.at[prev], wsem).wait()
            pltpu.make_async_copy(r4, o4.at[prev], wsem).wait()
        base = wid * RPW + ci * CHUNK
        pltpu.sync_copy(x_hbm.at[pl.ds(base * 5, CHUNK * 5)], xv)
        for g in range(G):
            for j in range(IDX_ROW // LANES):
                src = (g * IDX_ROW + j * LANES + lane) * 5
                sl = pl.ds(j * LANES, LANES)
                i0[g, sl] = lax.rem(plsc.load_gather(xv, [src]), VOC_BIG)
                i1[g, sl] = lax.rem(plsc.load_gather(xv, [src + 1]), VOC_BIG)
                i4[g, sl] = lax.rem(plsc.load_gather(xv, [src + 4]), VOC_4)
        cps = []
        for g in range(G):
            dst = pl.ds(g * IDX_ROW, IDX_ROW)
            cps.append(pltpu.async_copy(e0.at[i0.at[g]], r0.at[dst], gsem))
            cps.append(pltpu.async_copy(e1.at[i1.at[g]], r1.at[dst], gsem))
            cps.append(pltpu.async_copy(e4.at[i4.at[g]], r4.at[dst], gsem))
        for c in cps:
            c.wait()
        rows = pl.ds(base, CHUNK)
        pltpu.async_copy(r0, o0.at[rows], wsem)
        pltpu.async_copy(r1, o1.at[rows], wsem)
        pltpu.async_copy(r4, o4.at[rows], wsem)

    def pair_body(k, carry):
        one_chunk(2 * k, 0, k > 0)
        one_chunk(2 * k + 1, 1, k > 0)
        return carry

    lax.fori_loop(0, NCHUNK // 2, pair_body, 0)
    for ci in (NCHUNK - 2, NCHUNK - 1):
        r0, r1, r4 = rbufs[ci % 2]
        last = pl.ds(wid * RPW + ci * CHUNK, CHUNK)
        pltpu.make_async_copy(r0, o0.at[last], wsem).wait()
        pltpu.make_async_copy(r1, o1.at[last], wsem).wait()
        pltpu.make_async_copy(r4, o4.at[last], wsem).wait()


def _sc_gather(xflat, emb0, emb1, emb4):
    mesh = plsc.VectorSubcoreMesh(core_axis_name="c", subcore_axis_name="s")
    out = jax.ShapeDtypeStruct((N, D), jnp.bfloat16)
    rbuf = pltpu.VMEM((CHUNK, D), jnp.bfloat16)
    run = pl.kernel(
        _sc_gather_body,
        out_type=(out, out, out),
        mesh=mesh,
        compiler_params=pltpu.CompilerParams(
            needs_layout_passes=False, use_tc_tiling_on_sc=False),
        scratch_types=[
            pltpu.VMEM((CHUNK * 5,), jnp.int32),
            pltpu.VMEM((G, IDX_ROW), jnp.int32),
            pltpu.VMEM((G, IDX_ROW), jnp.int32),
            pltpu.VMEM((G, IDX_ROW), jnp.int32),
            ((rbuf, rbuf, rbuf), (rbuf, rbuf, rbuf)),
            pltpu.SemaphoreType.DMA,
            pltpu.SemaphoreType.DMA,
        ],
    )
    return run(xflat, emb0, emb1, emb4)


TB = 512


def _elu(x):
    return jnp.where(x > 0, x, jnp.exp(jnp.minimum(x, 0.0)) - 1.0)


def _tc_dense_body(x_ref, g0_ref, g1_ref, g4_ref,
                   w0_ref, w1_ref, w4_ref, m23_ref, b1_ref,
                   w2_ref, b2_ref, s2_ref, t2_ref, o_ref):
    f32 = jnp.float32
    bf16 = jnp.bfloat16
    acc = jnp.dot(g0_ref[...], w0_ref[...], preferred_element_type=f32)
    acc += jnp.dot(g1_ref[...], w1_ref[...], preferred_element_type=f32)
    acc += jnp.dot(g4_ref[...], w4_ref[...], preferred_element_type=f32)
    x = x_ref[...]
    code = lax.rem(x[:, 2:3], 5) * 8 + lax.rem(x[:, 3:4], 8)
    oh = (code == lax.broadcasted_iota(jnp.int32, (TB, 40), 1)).astype(bf16)
    acc += jnp.dot(oh, m23_ref[...], preferred_element_type=f32)
    acc += b1_ref[...]
    a1 = _elu(acc).astype(bf16)
    z2 = jnp.dot(a1, w2_ref[...], preferred_element_type=f32) + b2_ref[...]
    o_ref[...] = _elu(z2) * s2_ref[...] + t2_ref[...]


def _tc_dense(x2d, g0, g1, g4, w0, w1, w4, m23, b1, w2, b2, s2, t2):
    row_spec = lambda c: pl.BlockSpec((TB, c), lambda i: (i, 0))
    full = lambda a: pl.BlockSpec(a.shape, lambda i: (0, 0))
    return pl.pallas_call(
        _tc_dense_body,
        grid=(N // TB,),
        in_specs=[
            row_spec(5), row_spec(D), row_spec(D), row_spec(D),
            full(w0), full(w1), full(w4), full(m23), full(b1),
            full(w2), full(b2), full(s2), full(t2),
        ],
        out_specs=row_spec(100),
        out_shape=jax.ShapeDtypeStruct((N, 100), jnp.float32),
        compiler_params=pltpu.CompilerParams(
            dimension_semantics=("arbitrary",)),
    )(x2d, g0, g1, g4, w0, w1, w4, m23, b1, w2, b2, s2, t2)


def kernel(X, emb0, emb1, emb2, emb3, emb4,
           gamma0, beta0, mmean0, mvar0,
           W1, bb1,
           gamma1, beta1, mmean1, mvar1,
           W2, bb2,
           gamma2, beta2, mmean2, mvar2):
    bf16 = jnp.bfloat16
    s0 = gamma0 * lax.rsqrt(mvar0 + 1e-3)
    t0 = beta0 - mmean0 * s0
    W1p = W1 * s0[:, None]
    b1p = t0 @ W1 + bb1
    s1 = gamma1 * lax.rsqrt(mvar1 + 1e-3)
    t1 = beta1 - mmean1 * s1
    W2p = W2 * s1[:, None]
    b2p = t1 @ W2 + bb2
    s2 = gamma2 * lax.rsqrt(mvar2 + 1e-3)
    t2 = beta2 - mmean2 * s2
    m23 = ((emb2 @ W1p[100:103])[:, None, :]
           + (emb3 @ W1p[103:107])[None, :, :]).reshape(40, 150)

    xflat = X.reshape(-1)
    x2d = X.reshape(N, 5)
    padt = lambda e: jnp.pad(e.astype(bf16), ((0, 0), (0, D - 50)))
    padw = lambda w: jnp.pad(w.astype(bf16), ((0, D - 50), (0, 0)))
    g0, g1, g4 = _sc_gather(xflat, padt(emb0), padt(emb1), padt(emb4))
    out = _tc_dense(
        x2d, g0, g1, g4,
        padw(W1p[0:50]), padw(W1p[50:100]), padw(W1p[107:157]),
        m23.astype(bf16), b1p.reshape(1, 150),
        W2p.astype(bf16), b2p.reshape(1, 100),
        s2.reshape(1, 100), t2.reshape(1, 100))
    return out.reshape(4096, 20, 100)


# speedup vs baseline: 1.3310x; 1.3310x over previous
import functools

import jax
import jax.numpy as jnp
from jax import lax
from jax.experimental import pallas as pl
from jax.experimental.pallas import tpu as pltpu
from jax.experimental.pallas import tpu_sc as plsc

N = 4096 * 20
NC, NS, LANES = 2, 16, 16
NW = NC * NS
RPW = N // NW
CHUNK = 256
IDX_ROW = 128
G = CHUNK // IDX_ROW
NCHUNK = RPW // CHUNK
D = 64
SD = 256
CCOL = 192
VOC_BIG = 100001
VOC_4 = 1001


def _sc_gather_body(x_hbm, e0, e1, e4, og, xv, i0, i1, i4, rbufs, gsem, wsem):
    wid = lax.axis_index("s") * NC + lax.axis_index("c")
    lane = lax.iota(jnp.int32, LANES)
    zero = jnp.zeros((LANES,), jnp.int32)

    def wb_list(ci, bufs):
        r0, r1, r4, rc = bufs
        rows = pl.ds(wid * RPW + ci * CHUNK, CHUNK)
        return [(r0, og.at[rows, pl.ds(0, D)]),
                (r1, og.at[rows, pl.ds(D, D)]),
                (r4, og.at[rows, pl.ds(2 * D, D)]),
                (rc, og.at[rows, pl.ds(CCOL, 16)])]

    def one_chunk(ci, p, guard):
        r0, r1, r4, rc = rbufs[p]
        @pl.when(guard)
        def _():
            for src, dst in wb_list(ci - 2, rbufs[p]):
                pltpu.make_async_copy(src, dst, wsem).wait()
        base = wid * RPW + ci * CHUNK
        pltpu.sync_copy(x_hbm.at[pl.ds(base * 5, CHUNK * 5)], xv)
        for g in range(G):
            for j in range(IDX_ROW // LANES):
                row = g * IDX_ROW + j * LANES
                src = (row + lane) * 5
                sl = pl.ds(j * LANES, LANES)
                i0[g, sl] = lax.rem(plsc.load_gather(xv, [src]), VOC_BIG)
                i1[g, sl] = lax.rem(plsc.load_gather(xv, [src + 1]), VOC_BIG)
                i4[g, sl] = lax.rem(plsc.load_gather(xv, [src + 4]), VOC_4)
                code = (lax.rem(plsc.load_gather(xv, [src + 2]), 5) * 8
                        + lax.rem(plsc.load_gather(xv, [src + 3]), 8))
                plsc.store_scatter(rc, [row + lane, zero],
                                   code.astype(jnp.float32))
        cps = []
        for g in range(G):
            dst = pl.ds(g * IDX_ROW, IDX_ROW)
            cps.append(pltpu.async_copy(e0.at[i0.at[g]], r0.at[dst], gsem))
            cps.append(pltpu.async_copy(e1.at[i1.at[g]], r1.at[dst], gsem))
            cps.append(pltpu.async_copy(e4.at[i4.at[g]], r4.at[dst], gsem))
        for c in cps:
            c.wait()
        for src, dst in wb_list(ci, rbufs[p]):
            pltpu.async_copy(src, dst, wsem)

    def pair_body(k, carry):
        one_chunk(2 * k, 0, k > 0)
        one_chunk(2 * k + 1, 1, k > 0)
        return carry

    lax.fori_loop(0, NCHUNK // 2, pair_body, 0)
    for ci in (NCHUNK - 2, NCHUNK - 1):
        for src, dst in wb_list(ci, rbufs[ci % 2]):
            pltpu.make_async_copy(src, dst, wsem).wait()


def _sc_gather(xflat, emb0, emb1, emb4):
    mesh = plsc.VectorSubcoreMesh(core_axis_name="c", subcore_axis_name="s")
    rbuf = pltpu.VMEM((CHUNK, D), jnp.float32)
    cbuf = pltpu.VMEM((CHUNK, 16), jnp.float32)
    run = pl.kernel(
        _sc_gather_body,
        out_type=jax.ShapeDtypeStruct((N, SD), jnp.float32),
        mesh=mesh,
        compiler_params=pltpu.CompilerParams(
            needs_layout_passes=False, use_tc_tiling_on_sc=False),
        scratch_types=[
            pltpu.VMEM((CHUNK * 5,), jnp.int32),
            pltpu.VMEM((G, IDX_ROW), jnp.int32),
            pltpu.VMEM((G, IDX_ROW), jnp.int32),
            pltpu.VMEM((G, IDX_ROW), jnp.int32),
            ((rbuf, rbuf, rbuf, cbuf), (rbuf, rbuf, rbuf, cbuf)),
            pltpu.SemaphoreType.DMA,
            pltpu.SemaphoreType.DMA,
        ],
    )
    return run(xflat, emb0, emb1, emb4)


TB = 640


def _elu(x):
    return jnp.where(x > 0, x, jnp.exp(x) - 1.0)


def _tc_dense_body(g_ref, w0_ref, w1_ref, w4_ref, m23_ref, b1_ref,
                   w2_ref, b2_ref, s2_ref, t2_ref, o_ref):
    f32 = jnp.float32
    bf16 = jnp.bfloat16
    gb = g_ref[...]
    acc = jnp.dot(gb[:, 0:D].astype(bf16), w0_ref[...],
                  preferred_element_type=f32)
    acc += jnp.dot(gb[:, D:2 * D].astype(bf16), w1_ref[...],
                   preferred_element_type=f32)
    acc += jnp.dot(gb[:, 2 * D:3 * D].astype(bf16), w4_ref[...],
                   preferred_element_type=f32)
    code = gb[:, CCOL:CCOL + 1].astype(jnp.int32)
    oh = (code == lax.broadcasted_iota(jnp.int32, (TB, 40), 1)).astype(bf16)
    acc += jnp.dot(oh, m23_ref[...], preferred_element_type=f32)
    acc += b1_ref[...]
    a1 = _elu(acc).astype(bf16)
    z2 = jnp.dot(a1, w2_ref[...], preferred_element_type=f32) + b2_ref[...]
    o_ref[...] = (_elu(z2) * s2_ref[...] + t2_ref[...]).reshape(o_ref.shape)


def _tc_dense(gbuf, w0, w1, w4, m23, b1, w2, b2, s2, t2):
    full = lambda a: pl.BlockSpec(a.shape, lambda i: (0,) * a.ndim)
    return pl.pallas_call(
        _tc_dense_body,
        grid=(N // TB,),
        in_specs=[
            pl.BlockSpec((TB, SD), lambda i: (i, 0)),
            full(w0), full(w1), full(w4), full(m23), full(b1),
            full(w2), full(b2), full(s2), full(t2),
        ],
        out_specs=pl.BlockSpec((TB // 20, 20, 100), lambda i: (i, 0, 0)),
        out_shape=jax.ShapeDtypeStruct((4096, 20, 100), jnp.float32),
        compiler_params=pltpu.CompilerParams(
            dimension_semantics=("arbitrary",)),
    )(gbuf, w0, w1, w4, m23, b1, w2, b2, s2, t2)


def kernel(X, emb0, emb1, emb2, emb3, emb4,
           gamma0, beta0, mmean0, mvar0,
           W1, bb1,
           gamma1, beta1, mmean1, mvar1,
           W2, bb2,
           gamma2, beta2, mmean2, mvar2):
    bf16 = jnp.bfloat16
    s0 = gamma0 * lax.rsqrt(mvar0 + 1e-3)
    t0 = beta0 - mmean0 * s0
    W1p = W1 * s0[:, None]
    b1p = t0 @ W1 + bb1
    s1 = gamma1 * lax.rsqrt(mvar1 + 1e-3)
    t1 = beta1 - mmean1 * s1
    W2p = W2 * s1[:, None]
    b2p = t1 @ W2 + bb2
    s2 = gamma2 * lax.rsqrt(mvar2 + 1e-3)
    t2 = beta2 - mmean2 * s2
    m23 = ((emb2 @ W1p[100:103])[:, None, :]
           + (emb3 @ W1p[103:107])[None, :, :]).reshape(40, 150)

    xflat = X.reshape(-1)
    padt = lambda e: jnp.pad(e, ((0, 0), (0, D - 50)))
    padw = lambda w: jnp.pad(w.astype(bf16), ((0, D - 50), (0, 0)))
    gbuf = _sc_gather(xflat, padt(emb0), padt(emb1), padt(emb4))
    return _tc_dense(
        gbuf,
        padw(W1p[0:50]), padw(W1p[50:100]), padw(W1p[107:157]),
        m23.astype(bf16), b1p.reshape(1, 150),
        W2p.astype(bf16), b2p.reshape(1, 100),
        s2.reshape(1, 100), t2.reshape(1, 100))


# speedup vs baseline: 1.4345x; 1.0777x over previous
import functools

import jax
import jax.numpy as jnp
from jax import lax
from jax.experimental import pallas as pl
from jax.experimental.pallas import tpu as pltpu
from jax.experimental.pallas import tpu_sc as plsc

N = 4096 * 20
NC, NS, LANES = 2, 16, 16
NW = NC * NS
RPW = N // NW
CHUNK = 128
IDX_ROW = 128
G = CHUNK // IDX_ROW
NCHUNK = RPW // CHUNK
D = 64
VOC_BIG = 100001
VOC_4 = 1001


def _sc_gather_body(x_hbm, e0, e1, e4, p0, p1, p4, pc,
                    xv, i0, i1, i4, rbufs, gsem, wsem):
    wid = lax.axis_index("s") * NC + lax.axis_index("c")
    lane = lax.iota(jnp.int32, LANES)
    zero = jnp.zeros((LANES,), jnp.int32)

    def wb_list(ci, bufs):
        r0, r1, r4, rc = bufs
        rows = pl.ds(wid * RPW + ci * CHUNK, CHUNK)
        return [(r0, p0.at[rows]), (r1, p1.at[rows]),
                (r4, p4.at[rows]), (rc, pc.at[rows])]

    def one_chunk(ci, p, guard):
        r0, r1, r4, rc = rbufs[p]
        @pl.when(guard)
        def _():
            for src, dst in wb_list(ci - 2, rbufs[p]):
                pltpu.make_async_copy(src, dst, wsem).wait()
        base = wid * RPW + ci * CHUNK
        pltpu.sync_copy(x_hbm.at[pl.ds(base * 5, CHUNK * 5)], xv)
        for g in range(G):
            for j in range(IDX_ROW // LANES):
                row = g * IDX_ROW + j * LANES
                src = (row + lane) * 5
                sl = pl.ds(j * LANES, LANES)
                i0[g, sl] = lax.rem(plsc.load_gather(xv, [src]), VOC_BIG)
                i1[g, sl] = lax.rem(plsc.load_gather(xv, [src + 1]), VOC_BIG)
                i4[g, sl] = lax.rem(plsc.load_gather(xv, [src + 4]), VOC_4)
                code = (lax.rem(plsc.load_gather(xv, [src + 2]), 5) * 8
                        + lax.rem(plsc.load_gather(xv, [src + 3]), 8))
                plsc.store_scatter(rc, [row + lane, zero],
                                   code.astype(jnp.float32))
        cps = []
        for g in range(G):
            dst = pl.ds(g * IDX_ROW, IDX_ROW)
            cps.append(pltpu.async_copy(e0.at[i0.at[g]], r0.at[dst], gsem))
            cps.append(pltpu.async_copy(e1.at[i1.at[g]], r1.at[dst], gsem))
            cps.append(pltpu.async_copy(e4.at[i4.at[g]], r4.at[dst], gsem))
        for c in cps:
            c.wait()
        for src, dst in wb_list(ci, rbufs[p]):
            pltpu.async_copy(src, dst, wsem)

    def pair_body(k, carry):
        one_chunk(2 * k, 0, k > 0)
        one_chunk(2 * k + 1, 1, k > 0)
        return carry

    lax.fori_loop(0, NCHUNK // 2, pair_body, 0)
    for ci in (NCHUNK - 2, NCHUNK - 1):
        for src, dst in wb_list(ci, rbufs[ci % 2]):
            pltpu.make_async_copy(src, dst, wsem).wait()


def _sc_gather(xflat, emb0, emb1, emb4):
    mesh = plsc.VectorSubcoreMesh(core_axis_name="c", subcore_axis_name="s")
    plane = jax.ShapeDtypeStruct((N, D), jnp.float32)
    rbuf = pltpu.VMEM((CHUNK, D), jnp.float32)
    run = pl.kernel(
        _sc_gather_body,
        out_type=(plane, plane, plane, plane),
        mesh=mesh,
        compiler_params=pltpu.CompilerParams(
            needs_layout_passes=False, use_tc_tiling_on_sc=False),
        scratch_types=[
            pltpu.VMEM((CHUNK * 5,), jnp.int32),
            pltpu.VMEM((G, IDX_ROW), jnp.int32),
            pltpu.VMEM((G, IDX_ROW), jnp.int32),
            pltpu.VMEM((G, IDX_ROW), jnp.int32),
            ((rbuf, rbuf, rbuf, rbuf), (rbuf, rbuf, rbuf, rbuf)),
            pltpu.SemaphoreType.DMA,
            pltpu.SemaphoreType.DMA,
        ],
    )
    return run(xflat, emb0, emb1, emb4)


TB = 640
TB2 = TB // 2
BS = TB // 20


def _elu(x):
    return jnp.where(x > 0, x, jnp.exp(x) - 1.0)


def _tc_dense_body(g0_ref, g1_ref, g4_ref, gc_ref,
                   w0_ref, w1_ref, w4_ref, m23l_ref, m23r_ref, b1_ref,
                   w2_ref, b2_ref, s2_ref, t2_ref, o_ref):
    f32 = jnp.float32
    bf16 = jnp.bfloat16
    acc = jnp.dot(g0_ref[...].astype(bf16), w0_ref[...],
                  preferred_element_type=f32)
    acc += jnp.dot(g1_ref[...].astype(bf16), w1_ref[...],
                   preferred_element_type=f32)
    acc += jnp.dot(g4_ref[...].astype(bf16), w4_ref[...],
                   preferred_element_type=f32)
    gc = gc_ref[...]
    ce = gc[:, 0:1].astype(jnp.int32)
    co = gc[:, D:D + 1].astype(jnp.int32)
    io40 = lax.broadcasted_iota(jnp.int32, (TB2, 40), 1)
    acc += jnp.dot((ce == io40).astype(bf16), m23l_ref[...],
                   preferred_element_type=f32)
    acc += jnp.dot((co == io40).astype(bf16), m23r_ref[...],
                   preferred_element_type=f32)
    acc += b1_ref[...]
    a1 = _elu(acc).astype(bf16)
    z2 = jnp.dot(a1, w2_ref[...], preferred_element_type=f32) + b2_ref[...]
    o_ref[...] = (_elu(z2) * s2_ref[...] + t2_ref[...]).reshape(o_ref.shape)


def _tc_dense(g0, g1, g4, gc, w0, w1, w4, m23l, m23r, b1, w2, b2, s2, t2):
    full = lambda a: pl.BlockSpec(a.shape, lambda i: (0,) * a.ndim)
    row_spec = pl.BlockSpec((TB2, 128), lambda i: (i, 0))
    return pl.pallas_call(
        _tc_dense_body,
        grid=(N // TB,),
        in_specs=[
            row_spec, row_spec, row_spec, row_spec,
            full(w0), full(w1), full(w4), full(m23l), full(m23r), full(b1),
            full(w2), full(b2), full(s2), full(t2),
        ],
        out_specs=pl.BlockSpec((BS, 10, 200), lambda i: (i, 0, 0)),
        out_shape=jax.ShapeDtypeStruct((4096, 10, 200), jnp.float32),
        compiler_params=pltpu.CompilerParams(
            dimension_semantics=("arbitrary",)),
    )(g0, g1, g4, gc, w0, w1, w4, m23l, m23r, b1, w2, b2, s2, t2)


def _fold2(w):
    z = jnp.zeros_like(w)
    return jnp.concatenate(
        [jnp.concatenate([w, z], axis=1), jnp.concatenate([z, w], axis=1)],
        axis=0)


def kernel(X, emb0, emb1, emb2, emb3, emb4,
           gamma0, beta0, mmean0, mvar0,
           W1, bb1,
           gamma1, beta1, mmean1, mvar1,
           W2, bb2,
           gamma2, beta2, mmean2, mvar2):
    bf16 = jnp.bfloat16
    s0 = gamma0 * lax.rsqrt(mvar0 + 1e-3)
    t0 = beta0 - mmean0 * s0
    W1p = W1 * s0[:, None]
    b1p = t0 @ W1 + bb1
    s1 = gamma1 * lax.rsqrt(mvar1 + 1e-3)
    t1 = beta1 - mmean1 * s1
    W2p = W2 * s1[:, None]
    b2p = t1 @ W2 + bb2
    s2 = gamma2 * lax.rsqrt(mvar2 + 1e-3)
    t2 = beta2 - mmean2 * s2
    m23 = ((emb2 @ W1p[100:103])[:, None, :]
           + (emb3 @ W1p[103:107])[None, :, :]).reshape(40, 150)
    mz = jnp.zeros_like(m23)

    xflat = X.reshape(-1)
    padt = lambda e: jnp.pad(e, ((0, 0), (0, D - 50)))
    padw = lambda w: _fold2(jnp.pad(w, ((0, D - 50), (0, 0)))).astype(bf16)
    dup = lambda v, m: jnp.concatenate([v, v]).reshape(1, 2 * m)
    planes = _sc_gather(xflat, padt(emb0), padt(emb1), padt(emb4))
    g0, g1, g4, gc = (p.reshape(N // 2, 128) for p in planes)
    out = _tc_dense(
        g0, g1, g4, gc,
        padw(W1p[0:50]), padw(W1p[50:100]), padw(W1p[107:157]),
        jnp.concatenate([m23, mz], axis=1).astype(bf16),
        jnp.concatenate([mz, m23], axis=1).astype(bf16),
        dup(b1p, 150),
        _fold2(W2p).astype(bf16), dup(b2p, 100), dup(s2, 100), dup(t2, 100))
    return out.reshape(4096, 20, 100)
